# Initial kernel scaffold; baseline (speedup 1.0000x reference)
import jax
import jax.numpy as jnp
from jax.experimental import pallas as pl


def kernel(output_0, output_1, target, dense_labels):
    raise NotImplementedError("write your pallas kernel here")


# speedup vs baseline: 1.6227x; 1.6227x over previous
import functools

import jax
import jax.numpy as jnp
from jax.experimental import pallas as pl
from jax.experimental.pallas import tpu as pltpu

_B = 4096
_C = 397
_D = 4096
_BT = 512


def _body(tgt_ref, o0_ref, x_ref, lab_ref, out_ref):
    i = pl.program_id(0)
    nsteps = pl.num_programs(0)

    o0 = o0_ref[...]
    tgt = tgt_ref[0, pl.ds(i * _BT, _BT)]
    m = jnp.max(o0, axis=-1, keepdims=True)
    lse = jnp.log(jnp.sum(jnp.exp(o0 - m), axis=-1)) + m[:, 0]
    cls_ids = jax.lax.broadcasted_iota(jnp.int32, (_BT, _C), 1)
    onehot = (cls_ids == tgt[:, None])
    picked = jnp.sum(jnp.where(onehot, o0, 0.0), axis=-1)
    ce_part = jnp.sum(lse - picked)

    x = x_ref[...]
    sp = jnp.maximum(x, 0.0) + jnp.log1p(jnp.exp(-jnp.abs(x)))
    sp_sum = jnp.sum(sp)
    t_rows = jnp.dot(onehot.astype(jnp.bfloat16), lab_ref[...],
                     preferred_element_type=jnp.float32)
    dot_sum = jnp.sum(x * t_rows)

    part = ce_part * (1.0 / _B) + (sp_sum - dot_sum) * (1.0 / (_B * _D))

    @pl.when(i == 0)
    def _init():
        out_ref[0, 0] = 0.0

    out_ref[0, 0] += part


@jax.jit
def kernel(output_0, output_1, target, dense_labels):
    grid = _B // _BT
    tgt2d = target.astype(jnp.int32).reshape(1, _B)
    lab_bf16 = dense_labels.astype(jnp.bfloat16)
    out = pl.pallas_call(
        _body,
        grid=(grid,),
        in_specs=[
            pl.BlockSpec((1, _B), lambda i: (0, 0)),
            pl.BlockSpec((_BT, _C), lambda i: (i, 0)),
            pl.BlockSpec((_BT, _D), lambda i: (i, 0)),
            pl.BlockSpec((_C, _D), lambda i: (0, 0)),
        ],
        out_specs=pl.BlockSpec(memory_space=pltpu.SMEM),
        out_shape=jax.ShapeDtypeStruct((1, 1), jnp.float32),
    )(tgt2d, output_0, output_1, lab_bf16)
    return out[0, 0]


# speedup vs baseline: 2.0191x; 1.2443x over previous
import jax
import jax.numpy as jnp
from jax.experimental import pallas as pl
from jax.experimental.pallas import tpu as pltpu

_B = 4096
_C = 397
_D = 4096
_BT = 512

_LN2 = 0.6931471805599453
_LOG2E = 1.4426950408889634


def _body(tgt_ref, o0_ref, x_ref, labt_ref, out_ref):
    i = pl.program_id(0)

    o0 = o0_ref[...]
    tgt = tgt_ref[0, pl.ds(i * _BT, _BT)]
    m = jnp.max(o0, axis=-1, keepdims=True)
    lse = jnp.log(jnp.sum(jnp.exp(o0 - m), axis=-1)) + m[:, 0]
    cls_ids = jax.lax.broadcasted_iota(jnp.int32, (_BT, _C), 1)
    onehot = (cls_ids == tgt[:, None])
    picked = jnp.sum(jnp.where(onehot, o0, 0.0), axis=-1)
    ce_part = jnp.sum(lse - picked)

    x = x_ref[...]
    max_sum = jnp.sum(jnp.maximum(x, 0.0))
    log_sum = jnp.sum(jnp.log2(1.0 + jnp.exp2(jnp.abs(x) * (-_LOG2E))))
    s = jnp.dot(x.astype(jnp.bfloat16), labt_ref[...],
                preferred_element_type=jnp.float32)
    dot_sum = jnp.sum(jnp.where(onehot, s, 0.0))

    part = (ce_part * (1.0 / _B)
            + (max_sum + _LN2 * log_sum - dot_sum) * (1.0 / (_B * _D)))

    @pl.when(i == 0)
    def _init():
        out_ref[0, 0] = 0.0

    out_ref[0, 0] += part


@jax.jit
def kernel(output_0, output_1, target, dense_labels):
    grid = _B // _BT
    tgt2d = target.astype(jnp.int32).reshape(1, _B)
    labt_bf16 = dense_labels.T.astype(jnp.bfloat16)
    out = pl.pallas_call(
        _body,
        grid=(grid,),
        in_specs=[
            pl.BlockSpec((1, _B), lambda i: (0, 0)),
            pl.BlockSpec((_BT, _C), lambda i: (i, 0)),
            pl.BlockSpec((_BT, _D), lambda i: (i, 0)),
            pl.BlockSpec((_D, _C), lambda i: (0, 0)),
        ],
        out_specs=pl.BlockSpec(memory_space=pltpu.SMEM),
        out_shape=jax.ShapeDtypeStruct((1, 1), jnp.float32),
    )(tgt2d, output_0, output_1, labt_bf16)
    return out[0, 0]
